# Initial kernel scaffold; baseline (speedup 1.0000x reference)
import jax
import jax.numpy as jnp
from jax.experimental import pallas as pl


def kernel(x, W1, W2, W3, L1, L2, L3, n1, n2, n3):
    raise NotImplementedError("write your pallas kernel here")


# speedup vs baseline: 3.5514x; 3.5514x over previous
import jax
import jax.numpy as jnp
from jax.experimental import pallas as pl
from jax.experimental.pallas import tpu as pltpu

_IMG = 784
_H = 2000
_K = 100
_DECAY = 0.8
_STEPS = 6
_BM = 64

_MIN32 = -2147483648
_MAX31 = 2147483647


def _dot_t(a, w):
    return jax.lax.dot_general(
        a, w, dimension_numbers=(((1,), (1,)), ((), ())),
        preferred_element_type=jnp.float32)


def _kwta(mem, k):
    min32 = jnp.int32(_MIN32)
    max31 = jnp.int32(_MAX31)
    b = jax.lax.bitcast_convert_type(mem, jnp.int32)
    v = b ^ (jax.lax.shift_right_arithmetic(b, 31) & max31)

    cnt_pos = jnp.sum((v >= 0).astype(jnp.int32), axis=-1, keepdims=True)
    c_u = jnp.where(cnt_pos >= k, min32, jnp.int32(0))

    def body(i, c_u):
        bit = jax.lax.shift_left(jnp.int32(1), 31 - i)
        trial_u = c_u | bit
        trial_v = trial_u ^ min32
        cnt = jnp.sum((v >= trial_v).astype(jnp.int32), axis=-1, keepdims=True)
        return jnp.where(cnt >= k, trial_u, c_u)

    c_u = jax.lax.fori_loop(1, 32, body, c_u)
    thr_v = c_u ^ min32

    mask = v >= thr_v
    new_mem = jnp.where(mask, mem + 0.5, 0.0)
    act = jnp.maximum(new_mem, 0.0)
    return new_mem, act


def _fused_kernel(cur1_ref, n2_ref, n3_ref, lc1_ref, lc2_ref, lc3_ref,
                  w2_ref, w3_ref,
                  out_ref,
                  m1_ref, m2_ref, m3_ref):
    f32 = jnp.float32
    zero = jnp.zeros((cur1_ref.shape[0], _H), dtype=f32)
    m1_ref[...] = zero
    m2_ref[...] = zero
    m3_ref[...] = zero

    act = zero
    for _ in range(_STEPS):
        mem = m1_ref[...] * _DECAY + cur1_ref[...]
        mem = mem + lc1_ref[...]
        mem = mem + 2.0
        m1, act = _kwta(mem, _K)
        m1_ref[...] = m1

        cur = _dot_t(act, w2_ref[...]) + n2_ref[...]
        mem = m2_ref[...] * _DECAY + cur
        mem = mem + lc2_ref[...]
        mem = mem + 2.0
        m2, act = _kwta(mem, _K)
        m2_ref[...] = m2

        cur = _dot_t(act, w3_ref[...]) + n3_ref[...]
        mem = m3_ref[...] * _DECAY + cur
        mem = mem + lc3_ref[...]
        mem = mem + 2.0
        m3, act = _kwta(mem, _K)
        m3_ref[...] = m3
    out_ref[...] = act


@jax.jit
def kernel(x, W1, W2, W3, L1, L2, L3, n1, n2, n3):
    B = x.shape[0]
    img = x[:, :_IMG]
    lbl = x[:, _IMG:]
    img = img / (jnp.linalg.norm(img, axis=1, keepdims=True) + 1e-08) * 30.0
    cur1 = img @ W1.T + n1
    lc1 = lbl @ L1.T
    lc2 = lbl @ L2.T
    lc3 = lbl @ L3.T

    nb = B // _BM
    row_spec = pl.BlockSpec((_BM, _H), lambda i: (i, 0))
    full_spec = pl.BlockSpec((_H, _H), lambda i: (0, 0))
    scratch = pltpu.VMEM((_BM, _H), jnp.float32)

    out = pl.pallas_call(
        _fused_kernel,
        grid=(nb,),
        in_specs=[row_spec] * 6 + [full_spec] * 2,
        out_specs=row_spec,
        out_shape=jax.ShapeDtypeStruct((B, _H), jnp.float32),
        scratch_shapes=[scratch] * 3,
        compiler_params=pltpu.CompilerParams(
            dimension_semantics=("arbitrary",),
        ),
    )(cur1, n2, n3, lc1, lc2, lc3, W2, W3)
    return out


# speedup vs baseline: 5.1978x; 1.4636x over previous
import jax
import jax.numpy as jnp
from jax.experimental import pallas as pl
from jax.experimental.pallas import tpu as pltpu

_IMG = 784
_H = 2000
_K = 100
_DECAY = 0.8
_STEPS = 6
_BM = 128

_MIN32 = -2147483648
_MAX31 = 2147483647


def _dot_t(a, w):
    return jax.lax.dot_general(
        a, w, dimension_numbers=(((1,), (1,)), ((), ())),
        preferred_element_type=jnp.float32)


def _kwta(mem, k):
    min32 = jnp.int32(_MIN32)
    max31 = jnp.int32(_MAX31)
    b = jax.lax.bitcast_convert_type(mem, jnp.int32)
    v = b ^ (jax.lax.shift_right_arithmetic(b, 31) & max31)

    cnt_pos = jnp.sum((v >= 0).astype(jnp.int32), axis=-1, keepdims=True)
    c_u = jnp.where(cnt_pos >= k, min32, jnp.int32(0))

    def body(i, c_u):
        bit = jax.lax.shift_left(jnp.int32(1), 31 - i)
        trial_u = c_u | bit
        trial_v = trial_u ^ min32
        cnt = jnp.sum((v >= trial_v).astype(jnp.int32), axis=-1, keepdims=True)
        return jnp.where(cnt >= k, trial_u, c_u)

    c_u = jax.lax.fori_loop(1, 32, body, c_u)
    thr_v = c_u ^ min32

    mask = v >= thr_v
    new_mem = jnp.where(mask, mem + 0.5, 0.0)
    act = jnp.maximum(new_mem, 0.0)
    return new_mem, act


def _fused_kernel(cur1_ref, n2_ref, n3_ref, lc1_ref, lc2_ref, lc3_ref,
                  w2_ref, w3_ref,
                  out_ref,
                  m1_ref, m2_ref, m3_ref):
    f32 = jnp.float32
    zero = jnp.zeros((cur1_ref.shape[0], _H), dtype=f32)
    m1_ref[...] = zero
    m2_ref[...] = zero
    m3_ref[...] = zero

    act = zero
    for _ in range(_STEPS):
        mem = m1_ref[...] * _DECAY + cur1_ref[...]
        mem = mem + lc1_ref[...]
        mem = mem + 2.0
        m1, act = _kwta(mem, _K)
        m1_ref[...] = m1

        cur = _dot_t(act, w2_ref[...]) + n2_ref[...]
        mem = m2_ref[...] * _DECAY + cur
        mem = mem + lc2_ref[...]
        mem = mem + 2.0
        m2, act = _kwta(mem, _K)
        m2_ref[...] = m2

        cur = _dot_t(act, w3_ref[...]) + n3_ref[...]
        mem = m3_ref[...] * _DECAY + cur
        mem = mem + lc3_ref[...]
        mem = mem + 2.0
        m3, act = _kwta(mem, _K)
        m3_ref[...] = m3
    out_ref[...] = act


@jax.jit
def kernel(x, W1, W2, W3, L1, L2, L3, n1, n2, n3):
    B = x.shape[0]
    img = x[:, :_IMG]
    lbl = x[:, _IMG:]
    img = img / (jnp.linalg.norm(img, axis=1, keepdims=True) + 1e-08) * 30.0
    cur1 = img @ W1.T + n1
    lc1 = lbl @ L1.T
    lc2 = lbl @ L2.T
    lc3 = lbl @ L3.T

    nb = B // _BM
    row_spec = pl.BlockSpec((_BM, _H), lambda i: (i, 0))
    full_spec = pl.BlockSpec((_H, _H), lambda i: (0, 0))
    scratch = pltpu.VMEM((_BM, _H), jnp.float32)

    out = pl.pallas_call(
        _fused_kernel,
        grid=(nb,),
        in_specs=[row_spec] * 6 + [full_spec] * 2,
        out_specs=row_spec,
        out_shape=jax.ShapeDtypeStruct((B, _H), jnp.float32),
        scratch_shapes=[scratch] * 3,
        compiler_params=pltpu.CompilerParams(
            dimension_semantics=("arbitrary",),
        ),
    )(cur1, n2, n3, lc1, lc2, lc3, W2, W3)
    return out
